# baseline (speedup 1.0000x reference)
import functools

import jax
import jax.numpy as jnp
from jax import lax
from jax.experimental import pallas as pl
from jax.experimental.pallas import tpu as pltpu
from jax.experimental.pallas import tpu_sc as plsc

BATCH = 16384
EMBED_DIM = 32
NW = 32
B_PER_W = BATCH // NW
CHUNK = 128
NCHUNK = B_PER_W // CHUNK
NGROUP = B_PER_W // 16


def _sc_kernel(users_h, product_h, utab_h, ptab_h, w_h, b_h, out_h,
               uidx, pidx, urows, prows, wv, bv, outv, sem):
    c = lax.axis_index("c")
    s = lax.axis_index("s")
    wid = s * 2 + c

    pltpu.sync_copy(users_h.at[wid], uidx)
    pltpu.sync_copy(product_h.at[wid], pidx)
    pltpu.sync_copy(w_h, wv)
    pltpu.sync_copy(b_h, bv)

    copies = []
    for j in range(NCHUNK):
        copies.append(pltpu.async_copy(
            utab_h.at[uidx.at[j]], urows.at[pl.ds(j * CHUNK, CHUNK)], sem))
    for j in range(NCHUNK):
        copies.append(pltpu.async_copy(
            ptab_h.at[pidx.at[j]], prows.at[pl.ds(j * CHUNK, CHUNK)], sem))
    for cp in copies:
        cp.wait()

    bias = bv[...]

    def group(g, carry):
        rowi = g * 16 + lax.iota(jnp.int32, 16)
        acc = bias
        for d in range(EMBED_DIM):
            cd = jnp.full((16,), d, jnp.int32)
            col = plsc.load_gather(urows, [rowi, cd])
            acc = acc + col * wv[d]
        for d in range(EMBED_DIM):
            cd = jnp.full((16,), d, jnp.int32)
            col = plsc.load_gather(prows, [rowi, cd])
            acc = acc + col * wv[EMBED_DIM + d]
        outv[pl.ds(g * 16, 16)] = acc
        return carry

    lax.fori_loop(0, NGROUP, group, 0)
    pltpu.sync_copy(outv, out_h.at[wid])


@jax.jit
def kernel(users, product, user_table, product_table, W, b):
    users_r = users.astype(jnp.int32).reshape(NW, NCHUNK, CHUNK)
    product_r = product.astype(jnp.int32).reshape(NW, NCHUNK, CHUNK)
    w_flat = jnp.broadcast_to(W.reshape(2 * EMBED_DIM)[:, None], (2 * EMBED_DIM, 16))
    b16 = jnp.broadcast_to(b.reshape(1), (16,))

    mesh = plsc.VectorSubcoreMesh(core_axis_name="c", subcore_axis_name="s")
    run = functools.partial(
        pl.kernel,
        mesh=mesh,
        compiler_params=pltpu.CompilerParams(
            needs_layout_passes=False, use_tc_tiling_on_sc=False),
        out_type=jax.ShapeDtypeStruct((NW, B_PER_W), jnp.float32),
        scratch_types=[
            pltpu.VMEM((NCHUNK, CHUNK), jnp.int32),
            pltpu.VMEM((NCHUNK, CHUNK), jnp.int32),
            pltpu.VMEM((B_PER_W, EMBED_DIM), jnp.float32),
            pltpu.VMEM((B_PER_W, EMBED_DIM), jnp.float32),
            pltpu.VMEM((2 * EMBED_DIM, 16), jnp.float32),
            pltpu.VMEM((16,), jnp.float32),
            pltpu.VMEM((B_PER_W,), jnp.float32),
            pltpu.SemaphoreType.DMA,
        ],
    )(_sc_kernel)
    out = run(users_r, product_r, user_table, product_table, w_flat, b16)
    return out.reshape(BATCH, 1)


# speedup vs baseline: 5.9110x; 5.9110x over previous
import functools

import jax
import jax.numpy as jnp
from jax import lax
from jax.experimental import pallas as pl
from jax.experimental.pallas import tpu as pltpu
from jax.experimental.pallas import tpu_sc as plsc

BATCH = 16384
EMBED_DIM = 32
NW = 32
B_PER_W = BATCH // NW
CHUNK = 128
NCHUNK = B_PER_W // CHUNK
BL = 16384


def _scores_body(w_ref, t_ref, o_ref):
    o_ref[...] = jnp.dot(
        w_ref[...], t_ref[...], preferred_element_type=jnp.float32)[0]


def _scores(w_half, t32):
    n = t32.shape[1]
    grid = (n + BL - 1) // BL
    return pl.pallas_call(
        _scores_body,
        grid=(grid,),
        in_specs=[
            pl.BlockSpec((1, EMBED_DIM), lambda j: (0, 0)),
            pl.BlockSpec((EMBED_DIM, BL), lambda j: (0, j)),
        ],
        out_specs=pl.BlockSpec((BL,), lambda j: (j,)),
        out_shape=jax.ShapeDtypeStruct((n,), jnp.float32),
    )(w_half, t32)


def _gather_body(su_h, sp_h, uidx_h, pidx_h, b_h, out_h,
                 uidx, pidx, uval, pval, bv, outv, sem):
    c = lax.axis_index("c")
    s = lax.axis_index("s")
    wid = s * 2 + c

    pltpu.sync_copy(uidx_h.at[wid], uidx)
    pltpu.sync_copy(pidx_h.at[wid], pidx)
    pltpu.sync_copy(b_h, bv)

    copies = []
    for j in range(NCHUNK):
        copies.append(pltpu.async_copy(su_h.at[uidx.at[j]], uval.at[j], sem))
    for j in range(NCHUNK):
        copies.append(pltpu.async_copy(sp_h.at[pidx.at[j]], pval.at[j], sem))
    for cp in copies:
        cp.wait()

    bias = bv[...]
    for k in range(B_PER_W // 16):
        j, o = divmod(k * 16, CHUNK)
        acc = uval[j, pl.ds(o, 16)] + pval[j, pl.ds(o, 16)] + bias
        outv[pl.ds(k * 16, 16)] = acc
    pltpu.sync_copy(outv, out_h.at[wid])


@jax.jit
def kernel(users, product, user_table, product_table, W, b):
    tu = jnp.transpose(user_table)
    tp = jnp.transpose(product_table)
    w = W.reshape(2 * EMBED_DIM)
    s_u = _scores(w[:EMBED_DIM].reshape(1, EMBED_DIM), tu)
    s_p = _scores(w[EMBED_DIM:].reshape(1, EMBED_DIM), tp)

    users_r = users.astype(jnp.int32).reshape(NW, NCHUNK, CHUNK)
    product_r = product.astype(jnp.int32).reshape(NW, NCHUNK, CHUNK)
    b16 = jnp.broadcast_to(b.reshape(1), (16,))

    mesh = plsc.VectorSubcoreMesh(core_axis_name="c", subcore_axis_name="s")
    run = functools.partial(
        pl.kernel,
        mesh=mesh,
        compiler_params=pltpu.CompilerParams(
            needs_layout_passes=False, use_tc_tiling_on_sc=False),
        out_type=jax.ShapeDtypeStruct((NW, B_PER_W), jnp.float32),
        scratch_types=[
            pltpu.VMEM((NCHUNK, CHUNK), jnp.int32),
            pltpu.VMEM((NCHUNK, CHUNK), jnp.int32),
            pltpu.VMEM((NCHUNK, CHUNK), jnp.float32),
            pltpu.VMEM((NCHUNK, CHUNK), jnp.float32),
            pltpu.VMEM((16,), jnp.float32),
            pltpu.VMEM((B_PER_W,), jnp.float32),
            pltpu.SemaphoreType.DMA,
        ],
    )(_gather_body)
    out = run(s_u, s_p, users_r, product_r, b16)
    return out.reshape(BATCH, 1)
